# Initial kernel scaffold; baseline (speedup 1.0000x reference)
import jax
import jax.numpy as jnp
from jax.experimental import pallas as pl


def kernel(x, emb):
    raise NotImplementedError("write your pallas kernel here")


# speedup vs baseline: 7.5601x; 7.5601x over previous
import jax
import jax.numpy as jnp
from jax.experimental import pallas as pl
from jax.experimental.pallas import tpu as pltpu

_BLK = 8192


def _body(brick_ref, rot_ref, emb_ref, o_ref):
    brick = brick_ref[...]
    rot = rot_ref[...]
    idx = (1 + brick) * (1 + rot // 90)
    onehot = (idx[:, None] == jax.lax.broadcasted_iota(jnp.int32, (_BLK, 8), 1)
              ).astype(jnp.float32)
    o_ref[...] = jnp.dot(onehot, emb_ref[...],
                         preferred_element_type=jnp.float32)


def kernel(x, emb):
    B, L, _ = x.shape
    dim = emb.shape[1]
    n = B * L
    xi = x.astype(jnp.int32)
    brick = xi[..., 0].reshape(n)
    rot = xi[..., 1].reshape(n)
    emb_p = jnp.zeros((8, dim), jnp.float32).at[:emb.shape[0]].set(emb)
    grid = (n // _BLK,)
    out = pl.pallas_call(
        _body,
        grid=grid,
        in_specs=[
            pl.BlockSpec((_BLK,), lambda i: (i,)),
            pl.BlockSpec((_BLK,), lambda i: (i,)),
            pl.BlockSpec((8, dim), lambda i: (0, 0)),
        ],
        out_specs=pl.BlockSpec((_BLK, dim), lambda i: (i, 0)),
        out_shape=jax.ShapeDtypeStruct((n, dim), jnp.float32),
    )(brick, rot, emb_p)
    return out.reshape(B, L, dim)
